# baseline (speedup 1.0000x reference)
import functools

import jax
import jax.numpy as jnp
from jax import lax
from jax.experimental import pallas as pl
from jax.experimental.pallas import tpu as pltpu
from jax.experimental.pallas import tpu_sc as plsc

B = 16384
D = 64
L = 16
NC = 2
NS = 16
NW = NC * NS
BPW = B // NW
CHUNK = 128
NCH = BPW // CHUNK


def _sqrt16(x):
    bits = plsc.bitcast(x, jnp.int32)
    y = plsc.bitcast(jnp.int32(0x5F3759DF) - (bits >> 1), jnp.float32)
    for _ in range(3):
        y = y * (1.5 - 0.5 * x * y * y)
    return x * y


@functools.partial(
    pl.kernel,
    mesh=plsc.VectorSubcoreMesh(core_axis_name="c", subcore_axis_name="s"),
    out_type=jax.ShapeDtypeStruct((B,), jnp.float32),
    compiler_params=pltpu.CompilerParams(
        needs_layout_passes=False, use_tc_tiling_on_sc=False),
    scratch_types=[
        pltpu.VMEM((BPW, 3), jnp.int32),
        pltpu.VMEM((NCH, CHUNK), jnp.int32),
        pltpu.VMEM((NCH, CHUNK), jnp.int32),
        pltpu.VMEM((NCH, CHUNK), jnp.int32),
        pltpu.VMEM((BPW, D), jnp.float32),
        pltpu.VMEM((BPW, D), jnp.float32),
        pltpu.VMEM((BPW, D), jnp.float32),
        pltpu.VMEM((BPW,), jnp.float32),
        pltpu.SemaphoreType.DMA,
    ],
)
def _transe_sc(tr_hbm, nodes_hbm, rel_hbm, out_hbm,
               tr_v, si_v, pi_v, oi_v, s_v, p_v, o_v, res_v, sem):
    wid = lax.axis_index("s") * NC + lax.axis_index("c")
    base = wid * BPW

    pltpu.sync_copy(tr_hbm.at[pl.ds(base, BPW)], tr_v)

    iota = lax.iota(jnp.int32, L)

    copies = []
    for j in range(NCH):
        for q in range(CHUNK // L):
            ri = iota + (j * CHUNK + q * L)
            sl = pl.ds(q * L, L)
            si_v[j, sl] = plsc.load_gather(tr_v, [ri, jnp.zeros((L,), jnp.int32)])
            pi_v[j, sl] = plsc.load_gather(tr_v, [ri, jnp.ones((L,), jnp.int32)])
            oi_v[j, sl] = plsc.load_gather(tr_v, [ri, jnp.full((L,), 2, jnp.int32)])
        dst = pl.ds(j * CHUNK, CHUNK)
        copies.append(pltpu.async_copy(nodes_hbm.at[si_v.at[j]], s_v.at[dst], sem))
        copies.append(pltpu.async_copy(rel_hbm.at[pi_v.at[j]], p_v.at[dst], sem))
        copies.append(pltpu.async_copy(nodes_hbm.at[oi_v.at[j]], o_v.at[dst], sem))
    for c in copies:
        c.wait()

    def lanesum(x):
        for shift in (8, 4, 2, 1):
            x = x + x.at[lax.iota(jnp.int32, L) ^ shift].get(
                mode="promise_in_bounds")
        return x

    lane = lax.iota(jnp.int32, L)

    def group_body(g, carry):
        out16 = jnp.zeros((L,), jnp.float32)
        for k in range(L):
            i = g * L + k
            acc = jnp.zeros((L,), jnp.float32)
            for c in range(D // L):
                sl = pl.ds(c * L, L)
                t = s_v[i, sl] + p_v[i, sl] - o_v[i, sl]
                acc = acc + t * t
            out16 = jnp.where(lane == k, lanesum(acc), out16)
        res_v[pl.ds(g * L, L)] = _sqrt16(out16)
        return carry

    lax.fori_loop(0, BPW // L, group_body, 0)

    pltpu.sync_copy(res_v, out_hbm.at[pl.ds(base, BPW)])


@jax.jit
def kernel(triples, nodes, relations):
    return _transe_sc(triples.astype(jnp.int32), nodes, relations)


# speedup vs baseline: 1.0729x; 1.0729x over previous
import functools

import jax
import jax.numpy as jnp
from jax import lax
from jax.experimental import pallas as pl
from jax.experimental.pallas import tpu as pltpu
from jax.experimental.pallas import tpu_sc as plsc

B = 16384
D = 64
L = 16
NC = 2
NS = 16
NW = NC * NS
BPW = B // NW
CHUNK = 128
NCH = BPW // CHUNK


def _sqrt16(x):
    bits = plsc.bitcast(x, jnp.int32)
    y = plsc.bitcast(jnp.int32(0x5F3759DF) - (bits >> 1), jnp.float32)
    for _ in range(3):
        y = y * (1.5 - 0.5 * x * y * y)
    return x * y


@functools.partial(
    pl.kernel,
    mesh=plsc.VectorSubcoreMesh(core_axis_name="c", subcore_axis_name="s"),
    out_type=jax.ShapeDtypeStruct((B,), jnp.float32),
    compiler_params=pltpu.CompilerParams(
        needs_layout_passes=False, use_tc_tiling_on_sc=False),
    scratch_types=[
        pltpu.VMEM((BPW,), jnp.int32),
        pltpu.VMEM((BPW,), jnp.int32),
        pltpu.VMEM((BPW,), jnp.int32),
        pltpu.VMEM((BPW, D), jnp.float32),
        pltpu.VMEM((BPW, D), jnp.float32),
        pltpu.VMEM((BPW, D), jnp.float32),
        pltpu.VMEM((BPW,), jnp.float32),
        pltpu.SemaphoreType.DMA,
    ],
)
def _transe_sc(si_hbm, pi_hbm, oi_hbm, nodes_hbm, rel_hbm, out_hbm,
               si_v, pi_v, oi_v, s_v, p_v, o_v, res_v, sem):
    wid = lax.axis_index("s") * NC + lax.axis_index("c")
    base = wid * BPW

    pltpu.sync_copy(si_hbm.at[pl.ds(base, BPW)], si_v)
    pltpu.sync_copy(pi_hbm.at[pl.ds(base, BPW)], pi_v)
    pltpu.sync_copy(oi_hbm.at[pl.ds(base, BPW)], oi_v)

    copies = []
    for j in range(NCH):
        src = pl.ds(j * CHUNK, CHUNK)
        dst = pl.ds(j * CHUNK, CHUNK)
        copies.append(pltpu.async_copy(nodes_hbm.at[si_v.at[src]], s_v.at[dst], sem))
        copies.append(pltpu.async_copy(rel_hbm.at[pi_v.at[src]], p_v.at[dst], sem))
        copies.append(pltpu.async_copy(nodes_hbm.at[oi_v.at[src]], o_v.at[dst], sem))
    for c in copies:
        c.wait()

    def lanesum(x):
        for shift in (8, 4, 2, 1):
            x = x + x.at[lax.iota(jnp.int32, L) ^ shift].get(
                mode="promise_in_bounds")
        return x

    lane = lax.iota(jnp.int32, L)

    def group_body(g, carry):
        out16 = jnp.zeros((L,), jnp.float32)
        for k in range(L):
            i = g * L + k
            acc = jnp.zeros((L,), jnp.float32)
            for c in range(D // L):
                sl = pl.ds(c * L, L)
                t = s_v[i, sl] + p_v[i, sl] - o_v[i, sl]
                acc = acc + t * t
            out16 = jnp.where(lane == k, lanesum(acc), out16)
        res_v[pl.ds(g * L, L)] = _sqrt16(out16)
        return carry

    lax.fori_loop(0, BPW // L, group_body, 0)

    pltpu.sync_copy(res_v, out_hbm.at[pl.ds(base, BPW)])


@jax.jit
def kernel(triples, nodes, relations):
    t = triples.astype(jnp.int32)
    return _transe_sc(t[:, 0], t[:, 1], t[:, 2], nodes, relations)
